# baseline (speedup 1.0000x reference)
import math

import jax
import jax.numpy as jnp
from jax import lax
from jax.experimental import pallas as pl
from jax.experimental.pallas import tpu as pltpu
from jax.experimental.pallas import tpu_sc as plsc

VOCAB = 100000
N_EMBD = 64
BUCKET = 8191
NUM_TABLES = 3
B, T = 1024, 200
N_TOKENS = B * T

NUM_CORES = 2
NUM_SUBCORES = 16
LANES = 16
NW = NUM_CORES * NUM_SUBCORES
ROWS_PER_W = B // NW
CHUNK = T
NCHUNK = ROWS_PER_W
SEG0 = 128
SEG1 = CHUNK - SEG0
META = NUM_TABLES * CHUNK
SCALE = math.sqrt(N_EMBD)


def _splat(v):
    return jnp.full((LANES,), v, jnp.int32)


def _make_lookup():
    mesh = plsc.VectorSubcoreMesh(core_axis_name="c", subcore_axis_name="s")

    def body(x_hbm, ai_hbm, imp_hbm, tab_hbm, out_hbm,
             x_v, w_v, idx_v, rows_v, out_v,
             x_sem, meta_sem, rows_sem, out_sem):
        wid = lax.axis_index("s") * NUM_CORES + lax.axis_index("c")

        def x_copies(k, px):
            base = (wid * ROWS_PER_W + k) * CHUNK
            return (pltpu.make_async_copy(
                        x_hbm.at[pl.ds(base, SEG0)], x_v.at[px, 0],
                        x_sem.at[px]),
                    pltpu.make_async_copy(
                        x_hbm.at[pl.ds(base + SEG0, SEG1)],
                        x_v.at[px, 1, pl.ds(0, SEG1)], x_sem.at[px]))

        def fire_x(k, px):
            for c in x_copies(k, px):
                c.start()

        def wait_x(k, px):
            for c in x_copies(k, px):
                c.wait()

        def meta_copies(px, pm):
            xsegs = ((x_v.at[px, 0], 0, SEG0),
                     (x_v.at[px, 1, pl.ds(0, SEG1)], SEG0, SEG1))
            cs = []
            for i in range(NUM_TABLES):
                for xseg, off, n in xsegs:
                    cs.append(pltpu.make_async_copy(
                        ai_hbm.at[i].at[xseg],
                        idx_v.at[pm, i, pl.ds(off, n)], meta_sem.at[pm]))
                    cs.append(pltpu.make_async_copy(
                        imp_hbm.at[i].at[xseg],
                        w_v.at[pl.ds(pm * META + i * CHUNK + off, n)],
                        meta_sem.at[pm]))
            return cs

        def fire_meta(px, pm):
            for c in meta_copies(px, pm):
                c.start()

        def wait_meta(px, pm):
            for c in meta_copies(px, pm):
                c.wait()

        def rows_copies(pm, pr):
            cs = []
            for i in range(NUM_TABLES):
                for off, n in ((0, SEG0), (SEG0, SEG1)):
                    cs.append(pltpu.make_async_copy(
                        tab_hbm.at[i].at[idx_v.at[pm, i, pl.ds(off, n)]],
                        rows_v.at[pr, i, pl.ds(off, n)], rows_sem.at[pr]))
            return cs

        def fire_rows(pm, pr):
            for c in rows_copies(pm, pr):
                c.start()

        def wait_rows(pm, pr):
            for c in rows_copies(pm, pr):
                c.wait()

        def drain_out(pr):
            pltpu.make_async_copy(out_v.at[pr], out_hbm.at[0],
                                  out_sem.at[pr]).wait()

        def compute(k, pm, pr):
            @plsc.parallel_loop(0, CHUNK, step=1, unroll=8)
            def tok_body(t):
                wbase = pm * META + t
                w0 = plsc.load_gather(w_v, [_splat(wbase)]) * SCALE
                w1 = plsc.load_gather(w_v, [_splat(wbase + CHUNK)]) * SCALE
                w2 = plsc.load_gather(w_v, [_splat(wbase + 2 * CHUNK)]) * SCALE
                for q in range(N_EMBD // LANES):
                    sl = pl.ds(q * LANES, LANES)
                    acc = (w0 * rows_v[pr, 0, t, sl]
                           + w1 * rows_v[pr, 1, t, sl]
                           + w2 * rows_v[pr, 2, t, sl])
                    out_v[pr, t, sl] = acc

            pltpu.async_copy(out_v.at[pr], out_hbm.at[wid * ROWS_PER_W + k],
                             out_sem.at[pr])

        fire_x(0, 0)
        fire_x(1, 1)
        wait_x(0, 0)
        fire_meta(0, 0)
        wait_x(1, 1)
        fire_meta(1, 1)
        wait_meta(0, 0)
        fire_rows(0, 0)
        fire_x(2, 0)

        def chunk_quad(kk, carry):
            for j in range(4):
                k = kk * 4 + j
                pm, pr, px = k % 4, k % 2, k % 2

                @pl.when(k + 1 < NCHUNK)
                def _():
                    wait_meta((k + 1) % 2, (k + 1) % 4)
                    fire_rows((k + 1) % 4, (k + 1) % 2)

                @pl.when(k + 2 < NCHUNK)
                def _():
                    wait_x(k + 2, (k + 2) % 2)
                    fire_meta((k + 2) % 2, (k + 2) % 4)

                @pl.when(k + 3 < NCHUNK)
                def _():
                    fire_x(k + 3, (k + 3) % 2)

                wait_rows(pm, pr)

                @pl.when(k >= 2)
                def _():
                    drain_out(pr)

                compute(k, pm, pr)
            return carry

        lax.fori_loop(0, NCHUNK // 4, chunk_quad, 0)
        drain_out(0)
        drain_out(1)

    return pl.kernel(
        body,
        out_type=jax.ShapeDtypeStruct((B, T, N_EMBD), jnp.float32),
        mesh=mesh,
        compiler_params=pltpu.CompilerParams(
            needs_layout_passes=False, use_tc_tiling_on_sc=False),
        scratch_types=[
            pltpu.VMEM((2, 2, SEG0), jnp.int32),
            pltpu.VMEM((4 * META,), jnp.float32),
            pltpu.VMEM((4, NUM_TABLES, CHUNK), jnp.int32),
            pltpu.VMEM((2, NUM_TABLES, CHUNK, N_EMBD), jnp.float32),
            pltpu.VMEM((2, CHUNK, N_EMBD), jnp.float32),
            pltpu.SemaphoreType.DMA((2,)),
            pltpu.SemaphoreType.DMA((4,)),
            pltpu.SemaphoreType.DMA((2,)),
            pltpu.SemaphoreType.DMA((2,)),
        ],
    )


def kernel(x, all_indices, tables, importance):
    x_flat = x.reshape(-1)
    ai_t = all_indices.T
    imp_t = importance.T
    lookup = _make_lookup()
    return lookup(x_flat, ai_t, imp_t, tables)


# speedup vs baseline: 1.0102x; 1.0102x over previous
import math

import jax
import jax.numpy as jnp
from jax import lax
from jax.experimental import pallas as pl
from jax.experimental.pallas import tpu as pltpu
from jax.experimental.pallas import tpu_sc as plsc

VOCAB = 100000
N_EMBD = 64
BUCKET = 8191
NUM_TABLES = 3
B, T = 1024, 200
N_TOKENS = B * T

NUM_CORES = 2
NUM_SUBCORES = 16
LANES = 16
NW = NUM_CORES * NUM_SUBCORES
ROWS_PER_W = B // NW
CHUNK = T
NCHUNK = ROWS_PER_W
SEG0 = 128
SEG1 = CHUNK - SEG0
META = NUM_TABLES * CHUNK
SCALE = math.sqrt(N_EMBD)


def _splat(v):
    return jnp.full((LANES,), v, jnp.int32)


def _make_lookup():
    mesh = plsc.VectorSubcoreMesh(core_axis_name="c", subcore_axis_name="s")

    def body(x_hbm, ai_hbm, imp_hbm, tab_hbm, out_hbm,
             x_v, w_v, idx_v, rows_v, out_v,
             x_sem, meta_sem, rows_sem, out_sem):
        wid = lax.axis_index("s") * NUM_CORES + lax.axis_index("c")

        def x_copies(k, px):
            base = (wid * ROWS_PER_W + k) * CHUNK
            return (pltpu.make_async_copy(
                        x_hbm.at[pl.ds(base, SEG0)], x_v.at[px, 0],
                        x_sem.at[px]),
                    pltpu.make_async_copy(
                        x_hbm.at[pl.ds(base + SEG0, SEG1)],
                        x_v.at[px, 1, pl.ds(0, SEG1)], x_sem.at[px]))

        def fire_x(k, px):
            for c in x_copies(k, px):
                c.start()

        def wait_x(k, px):
            for c in x_copies(k, px):
                c.wait()

        def meta_copies(px, pm):
            xsegs = ((x_v.at[px, 0], 0, SEG0),
                     (x_v.at[px, 1, pl.ds(0, SEG1)], SEG0, SEG1))
            cs = []
            for i in range(NUM_TABLES):
                for xseg, off, n in xsegs:
                    cs.append(pltpu.make_async_copy(
                        ai_hbm.at[i].at[xseg],
                        idx_v.at[pm, i, pl.ds(off, n)], meta_sem.at[pm]))
                    cs.append(pltpu.make_async_copy(
                        imp_hbm.at[i].at[xseg],
                        w_v.at[pl.ds(pm * META + i * CHUNK + off, n)],
                        meta_sem.at[pm]))
            return cs

        def fire_meta(px, pm):
            for c in meta_copies(px, pm):
                c.start()

        def wait_meta(px, pm):
            for c in meta_copies(px, pm):
                c.wait()

        def rows_copies(pm, pr):
            cs = []
            for i in range(NUM_TABLES):
                for off, n in ((0, SEG0), (SEG0, SEG1)):
                    cs.append(pltpu.make_async_copy(
                        tab_hbm.at[i].at[idx_v.at[pm, i, pl.ds(off, n)]],
                        rows_v.at[pr, i, pl.ds(off, n)], rows_sem.at[pr]))
            return cs

        def fire_rows(pm, pr):
            for c in rows_copies(pm, pr):
                c.start()

        def wait_rows(pm, pr):
            for c in rows_copies(pm, pr):
                c.wait()

        def drain_out(pr):
            pltpu.make_async_copy(out_v.at[pr], out_hbm.at[0],
                                  out_sem.at[pr]).wait()

        def compute(k, pm, pr):
            @plsc.parallel_loop(0, CHUNK, step=1, unroll=4)
            def tok_body(t):
                wbase = pm * META + t
                w0 = plsc.load_gather(w_v, [_splat(wbase)]) * SCALE
                w1 = plsc.load_gather(w_v, [_splat(wbase + CHUNK)]) * SCALE
                w2 = plsc.load_gather(w_v, [_splat(wbase + 2 * CHUNK)]) * SCALE
                for q in range(N_EMBD // LANES):
                    sl = pl.ds(q * LANES, LANES)
                    acc = (w0 * rows_v[pr, 0, t, sl]
                           + w1 * rows_v[pr, 1, t, sl]
                           + w2 * rows_v[pr, 2, t, sl])
                    out_v[pr, t, sl] = acc

            pltpu.async_copy(out_v.at[pr], out_hbm.at[wid * ROWS_PER_W + k],
                             out_sem.at[pr])

        fire_x(0, 0)
        fire_x(1, 1)
        wait_x(0, 0)
        fire_meta(0, 0)
        wait_x(1, 1)
        fire_meta(1, 1)
        wait_meta(0, 0)
        fire_rows(0, 0)
        fire_x(2, 0)

        def chunk_quad(kk, carry):
            for j in range(4):
                k = kk * 4 + j
                pm, pr, px = k % 4, k % 2, k % 2

                @pl.when(k + 1 < NCHUNK)
                def _():
                    wait_meta((k + 1) % 2, (k + 1) % 4)
                    fire_rows((k + 1) % 4, (k + 1) % 2)

                @pl.when(k + 2 < NCHUNK)
                def _():
                    wait_x(k + 2, (k + 2) % 2)
                    fire_meta((k + 2) % 2, (k + 2) % 4)

                @pl.when(k + 3 < NCHUNK)
                def _():
                    fire_x(k + 3, (k + 3) % 2)

                wait_rows(pm, pr)

                @pl.when(k >= 2)
                def _():
                    drain_out(pr)

                compute(k, pm, pr)
            return carry

        lax.fori_loop(0, NCHUNK // 4, chunk_quad, 0)
        drain_out(0)
        drain_out(1)

    return pl.kernel(
        body,
        out_type=jax.ShapeDtypeStruct((B, T, N_EMBD), jnp.float32),
        mesh=mesh,
        compiler_params=pltpu.CompilerParams(
            needs_layout_passes=False, use_tc_tiling_on_sc=False),
        scratch_types=[
            pltpu.VMEM((2, 2, SEG0), jnp.int32),
            pltpu.VMEM((4 * META,), jnp.float32),
            pltpu.VMEM((4, NUM_TABLES, CHUNK), jnp.int32),
            pltpu.VMEM((2, NUM_TABLES, CHUNK, N_EMBD), jnp.float32),
            pltpu.VMEM((2, CHUNK, N_EMBD), jnp.float32),
            pltpu.SemaphoreType.DMA((2,)),
            pltpu.SemaphoreType.DMA((4,)),
            pltpu.SemaphoreType.DMA((2,)),
            pltpu.SemaphoreType.DMA((2,)),
        ],
    )


def kernel(x, all_indices, tables, importance):
    x_flat = x.reshape(-1)
    ai_t = all_indices.T
    imp_t = importance.T
    lookup = _make_lookup()
    return lookup(x_flat, ai_t, imp_t, tables)
